# Initial kernel scaffold; baseline (speedup 1.0000x reference)
import jax
import jax.numpy as jnp
from jax.experimental import pallas as pl


def kernel(x, edge_attr, edge_index, batch_index, params):
    raise NotImplementedError("write your pallas kernel here")


# speedup vs baseline: 26.1331x; 26.1331x over previous
import functools
import math

import jax
import jax.numpy as jnp
from jax import lax
from jax.experimental import pallas as pl
from jax.experimental.pallas import tpu as pltpu
from jax.experimental.pallas import tpu_sc as plsc

N = 50000
E = 800000
VALUE_DIM = 1000
FEAT = 128
EMB = 32
HEADS = 2
HC = HEADS * EMB
NG = 64
N2 = 51200
NR = 50304
RBN = 1048
NC = 2
NS = 16
NW = NC * NS
EB = 64
NBLK = 782
E_TILE = EB * NBLK
E2 = E_TILE * NS
ACC_W = 33


def _mesh():
    return plsc.VectorSubcoreMesh(
        core_axis_name="c", subcore_axis_name="s", num_cores=NC,
        num_subcores=NS)


@functools.partial(
    pl.kernel,
    out_type=jax.ShapeDtypeStruct((N2, FEAT), jnp.float32),
    mesh=_mesh(),
    scratch_types=[
        pltpu.VMEM((200,), jnp.int32),
        pltpu.VMEM((200, FEAT), jnp.float32),
        pltpu.SemaphoreType.DMA,
    ],
)
def _emb_gather(emb_hbm, x_hbm, out_hbm, idx_v, rows_v, sem):
    c = lax.axis_index("c")
    s = lax.axis_index("s")
    wid = s * NC + c
    rows_w = N2 // NW

    def blk(i, carry):
        base = pl.multiple_of(wid * rows_w + i * 200, 8)
        pltpu.sync_copy(x_hbm.at[pl.ds(base, 200)], idx_v)
        pltpu.async_copy(emb_hbm.at[idx_v], rows_v, sem).wait()
        pltpu.sync_copy(rows_v, out_hbm.at[pl.ds(base, 200)])
        return carry

    lax.fori_loop(0, rows_w // 200, blk, 0)


def _pre(h, scale, shift, wcat, bcat):
    din = h.shape[1]
    grid = NR // RBN

    def body(h_ref, sc_ref, sh_ref, w_ref, b_ref, tab_ref, xr_ref):
        hb = h_ref[...] * sc_ref[...] + sh_ref[...]
        o = jnp.dot(hb, w_ref[...], preferred_element_type=jnp.float32)
        o = o + b_ref[...]
        tab_ref[0] = o[:, 0:128]
        tab_ref[1] = o[:, 128:256]
        xr_ref[...] = o[:, 256:320]

    return pl.pallas_call(
        body,
        grid=(grid,),
        in_specs=[
            pl.BlockSpec((RBN, din), lambda i: (i, 0)),
            pl.BlockSpec((1, din), lambda i: (0, 0)),
            pl.BlockSpec((1, din), lambda i: (0, 0)),
            pl.BlockSpec((din, 320), lambda i: (0, 0)),
            pl.BlockSpec((1, 320), lambda i: (0, 0)),
        ],
        out_specs=[
            pl.BlockSpec((2, RBN, 128), lambda i: (0, i, 0)),
            pl.BlockSpec((RBN, 64), lambda i: (i, 0)),
        ],
        out_shape=[
            jax.ShapeDtypeStruct((2, NR, 128), jnp.float32),
            jax.ShapeDtypeStruct((NR, 64), jnp.float32),
        ],
    )(h, scale, shift, wcat, bcat)


def _emat(a4, wp):
    RB = 1088
    grid = (E2 // 4) // RB

    def body(a_ref, w_ref, e_ref):
        a = a_ref[...]
        e_ref[0] = jnp.dot(a, w_ref[0], preferred_element_type=jnp.float32)
        e_ref[1] = jnp.dot(a, w_ref[1], preferred_element_type=jnp.float32)

    return pl.pallas_call(
        body,
        grid=(grid,),
        in_specs=[
            pl.BlockSpec((RB, 64), lambda i: (i, 0)),
            pl.BlockSpec((2, 64, 128), lambda i: (0, 0, 0)),
        ],
        out_specs=pl.BlockSpec((2, RB, 128), lambda i: (0, i, 0)),
        out_shape=jax.ShapeDtypeStruct((2, E2 // 4, 128), jnp.float32),
    )(a4, wp)


@functools.partial(
    pl.kernel,
    out_type=jax.ShapeDtypeStruct((2, NR, ACC_W), jnp.float32),
    mesh=_mesh(),
    scratch_types=[
        pltpu.VMEM((EB,), jnp.int32),
        pltpu.VMEM((EB,), jnp.int32),
        pltpu.VMEM((EB,), jnp.int32),
        pltpu.VMEM((EB, 128), jnp.float32),
        pltpu.VMEM((EB, 128), jnp.float32),
        pltpu.VMEM((EB // 4, 128), jnp.float32),
        pltpu.VMEM((EB, ACC_W), jnp.float32),
        pltpu.VMEM_SHARED((NR, ACC_W), jnp.float32),
        pltpu.VMEM((24,), jnp.int32),
        pltpu.SemaphoreType.DMA,
        pltpu.SemaphoreType.DMA,
    ],
)
def _edge(tab_hbm, e_hbm, src_hbm, dst_hbm, zeros_hbm, out_hbm,
          idx_s, idx_d, idx_dq, kv_v, q_v, e_v, msg_v, acc, idx_z, sem1, sem2):
    c = lax.axis_index("c")
    s = lax.axis_index("s")
    rows_t = NR // NS
    lane = lax.iota(jnp.int32, 16)
    pltpu.sync_copy(zeros_hbm.at[pl.ds(0, 24)], msg_v.at[pl.ds(0, 24)])

    def zacc(i, carry):
        row = s * rows_t + i * 24
        idx_z[pl.ds(0, 16)] = jnp.full((16,), row, jnp.int32) + lane
        idx_z[pl.ds(8, 16)] = jnp.full((16,), row + 8, jnp.int32) + lane
        pltpu.sync_copy(msg_v.at[pl.ds(0, 24)], acc.at[idx_z])
        return carry

    lax.fori_loop(0, rows_t // 24, zacc, 0)
    plsc.subcore_barrier()

    rowoff = c * NR
    m1 = lax.max(lane - 14, 0).astype(jnp.float32)
    m0 = 1.0 - m1

    def blk(i, carry):
        base = pl.multiple_of(s * E_TILE + i * EB, 32)
        pltpu.sync_copy(src_hbm.at[pl.ds(base, EB)], idx_s)
        pltpu.sync_copy(dst_hbm.at[pl.ds(base, EB)], idx_d)

        def off(j, cy):
            sl = pl.ds(j * 16, 16)
            idx_s[sl] = idx_s[sl] + rowoff
            idx_dq[sl] = idx_d[sl] + rowoff
            return cy

        lax.fori_loop(0, EB // 16, off, 0)
        cp_kv = pltpu.async_copy(tab_hbm.at[idx_s], kv_v, sem1)
        cp_q = pltpu.async_copy(tab_hbm.at[idx_dq], q_v, sem2)
        pltpu.sync_copy(
            e_hbm.at[c, pl.ds(pl.multiple_of(base // 4, 8), EB // 4)], e_v)
        cp_kv.wait()
        cp_q.wait()

        def grp(r, cy):
            for u4 in range(4):
                j = r * 4 + u4
                k0 = kv_v[j, 0:16]
                k1 = kv_v[j, 16:32]
                v0 = kv_v[j, 32:48]
                v1 = kv_v[j, 48:64]
                q0 = q_v[j, 64:80]
                q1 = q_v[j, 80:96]
                ea = e_v[r, u4 * 32:u4 * 32 + 16]
                eb = e_v[r, u4 * 32 + 16:u4 * 32 + 32]
                t = q0 * (k0 + ea) + q1 * (k1 + eb)
                for shf in (8, 4, 2, 1):
                    t = t + t.at[lane ^ shf].get(mode="promise_in_bounds")
                exv = jnp.exp(t)
                msg_v[j, 0:16] = exv * (v0 + ea)
                msg_v[j, 16:32] = exv * (v1 + eb)
                old = msg_v[j, 17:33]
                msg_v[j, 17:33] = old * m0 + exv * m1
            return cy

        lax.fori_loop(0, EB // 4, grp, 0)
        pltpu.sync_copy(msg_v, acc.at[idx_d], add=True)
        return carry

    lax.fori_loop(0, NBLK, blk, 0)
    plsc.subcore_barrier()

    def flush(i, carry):
        row = s * rows_t + i * 24
        idx_z[pl.ds(0, 16)] = jnp.full((16,), row, jnp.int32) + lane
        idx_z[pl.ds(8, 16)] = jnp.full((16,), row + 8, jnp.int32) + lane
        pltpu.async_copy(acc.at[idx_z], msg_v.at[pl.ds(0, 24)], sem1).wait()
        pltpu.sync_copy(msg_v.at[pl.ds(0, 24)],
                        out_hbm.at[c, pl.ds(pl.multiple_of(row, 8), 24)])
        return carry

    lax.fori_loop(0, rows_t // 24, flush, 0)


def _post(acc3, xr, wa, wb, lw, lb, bn_g, bn_b):
    grid = NR // RBN

    def body(a_ref, xr_ref, wa_ref, wb_ref, lw_ref, lb_ref, g_ref, bb_ref,
             u_ref, sum_ref, ssq_ref, sc_ref, sh_ref):
        i = pl.program_id(0)
        a0 = a_ref[0]
        a1 = a_ref[1]
        out = jnp.concatenate(
            [a0[:, 0:32] / (a0[:, 32:33] + 1e-16),
             a1[:, 0:32] / (a1[:, 32:33] + 1e-16)], axis=1)
        xrb = xr_ref[...]
        blin = (jnp.sum(out * wa_ref[...], axis=1, keepdims=True)
                + jnp.sum(xrb * wb_ref[...], axis=1, keepdims=True))
        beta = jax.nn.sigmoid(blin)
        hmid = beta * xrb + (1.0 - beta) * out
        u = jnp.maximum(
            jnp.dot(hmid, lw_ref[...], preferred_element_type=jnp.float32)
            + lb_ref[...], 0.0)
        u_ref[...] = u
        rows = i * RBN + lax.broadcasted_iota(jnp.int32, (RBN, 1), 0)
        um = jnp.where(rows < N, u, 0.0)

        @pl.when(i == 0)
        def _():
            sum_ref[...] = jnp.zeros_like(sum_ref)
            ssq_ref[...] = jnp.zeros_like(ssq_ref)

        sum_ref[...] += jnp.sum(um, axis=0, keepdims=True)
        ssq_ref[...] += jnp.sum(um * um, axis=0, keepdims=True)

        @pl.when(i == grid - 1)
        def _():
            mean = sum_ref[...] / N
            var = ssq_ref[...] / N - mean * mean
            inv = lax.rsqrt(var + 1e-5)
            sc = g_ref[...] * inv
            sc_ref[...] = sc
            sh_ref[...] = bb_ref[...] - mean * sc

    return pl.pallas_call(
        body,
        grid=(grid,),
        in_specs=[
            pl.BlockSpec((2, RBN, ACC_W), lambda i: (0, i, 0)),
            pl.BlockSpec((RBN, 64), lambda i: (i, 0)),
            pl.BlockSpec((1, 64), lambda i: (0, 0)),
            pl.BlockSpec((1, 64), lambda i: (0, 0)),
            pl.BlockSpec((64, 32), lambda i: (0, 0)),
            pl.BlockSpec((1, 32), lambda i: (0, 0)),
            pl.BlockSpec((1, 32), lambda i: (0, 0)),
            pl.BlockSpec((1, 32), lambda i: (0, 0)),
        ],
        out_specs=[
            pl.BlockSpec((RBN, 32), lambda i: (i, 0)),
            pl.BlockSpec((1, 32), lambda i: (0, 0)),
            pl.BlockSpec((1, 32), lambda i: (0, 0)),
            pl.BlockSpec((1, 32), lambda i: (0, 0)),
            pl.BlockSpec((1, 32), lambda i: (0, 0)),
        ],
        out_shape=[
            jax.ShapeDtypeStruct((NR, 32), jnp.float32),
            jax.ShapeDtypeStruct((1, 32), jnp.float32),
            jax.ShapeDtypeStruct((1, 32), jnp.float32),
            jax.ShapeDtypeStruct((1, 32), jnp.float32),
            jax.ShapeDtypeStruct((1, 32), jnp.float32),
        ],
    )(acc3, xr, wa, wb, lw, lb, bn_g, bn_b)


def _poolk(u, batch_pad):
    grid = NR // RBN

    def body(u_ref, b_ref, gmx_ref, gsm_ref):
        i = pl.program_id(0)

        @pl.when(i == 0)
        def _():
            gmx_ref[...] = jnp.full_like(gmx_ref, -3.0e38)
            gsm_ref[...] = jnp.zeros_like(gsm_ref)

        ub = u_ref[...]
        bb = b_ref[...]
        glo = jnp.min(bb)
        ghi = jnp.minimum(jnp.max(bb), NG - 1)

        def upd(g, carry):
            m = (bb == g).astype(jnp.float32)
            bsum = jnp.sum(m * ub, axis=0, keepdims=True)
            bmax = jnp.max(jnp.where(m > 0, ub, -3.0e38), axis=0,
                           keepdims=True)
            sl = pl.ds(g, 1)
            gmx_ref[sl, :] = jnp.maximum(gmx_ref[sl, :], bmax)
            gsm_ref[sl, :] = gsm_ref[sl, :] + bsum
            return carry

        lax.fori_loop(glo, ghi + 1, upd, 0)

    return pl.pallas_call(
        body,
        grid=(grid,),
        in_specs=[
            pl.BlockSpec((RBN, 32), lambda i: (i, 0)),
            pl.BlockSpec((RBN, 1), lambda i: (i, 0)),
        ],
        out_specs=[
            pl.BlockSpec((NG, 32), lambda i: (0, 0)),
            pl.BlockSpec((NG, 32), lambda i: (0, 0)),
        ],
        out_shape=[
            jax.ShapeDtypeStruct((NG, 32), jnp.float32),
            jax.ShapeDtypeStruct((NG, 32), jnp.float32),
        ],
    )(u, batch_pad)


def _mlp(pools, cnt, w1, b1, w2, b2, w3, b3):
    flat = [r for t in pools for r in t]

    def body(*refs):
        (g1x, g1s, s1c, s1h, g2x, g2s, s2c, s2h, g3x, g3s, s3c, s3h,
         c_ref, w1_ref, b1_ref, w2_ref, b2_ref, w3_ref, b3_ref, o_ref) = refs
        cv = c_ref[...]
        pos = cv > 0.0
        cm = jnp.maximum(cv, 1.0)

        def rep_of(gx, gs, sc_, sh_):
            gmp = jnp.where(pos, gx[...] * sc_[...] + sh_[...], 0.0)
            gap = jnp.where(pos, (gs[...] * sc_[...]) / cm + sh_[...], 0.0)
            return jnp.concatenate([gmp, gap], axis=1)

        rep = (rep_of(g1x, g1s, s1c, s1h) + rep_of(g2x, g2s, s2c, s2h)
               + rep_of(g3x, g3s, s3c, s3h))
        z = jnp.maximum(
            jnp.dot(rep, w1_ref[...], preferred_element_type=jnp.float32)
            + b1_ref[...], 0.0)
        z = jnp.maximum(
            jnp.dot(z, w2_ref[...], preferred_element_type=jnp.float32)
            + b2_ref[...], 0.0)
        o = jnp.dot(z, w3_ref[...], preferred_element_type=jnp.float32)
        o_ref[...] = jax.nn.sigmoid(o + b3_ref[...])

    return pl.pallas_call(
        body,
        out_shape=jax.ShapeDtypeStruct((NG, 1), jnp.float32),
    )(*flat, cnt, w1, b1[None], w2, b2[None], w3, b3[None])


def _prep_conv(cp):
    s = 1.0 / math.sqrt(EMB)
    wq, wk, wv = cp["Wq"], cp["Wk"], cp["Wv"]
    din = wq.shape[0]
    z32 = jnp.zeros((din, 32), jnp.float32)
    wcat = jnp.concatenate(
        [wk[:, 0:32], wv[:, 0:32], wq[:, 0:32] * s, z32,
         wk[:, 32:64], wv[:, 32:64], wq[:, 32:64] * s, z32,
         cp["Wskip"]], axis=1)
    bq, bk, bv = cp["bq"], cp["bk"], cp["bv"]
    bz = jnp.zeros((32,), jnp.float32)
    bcat = jnp.concatenate(
        [bk[0:32], bv[0:32], bq[0:32] * s, bz,
         bk[32:64], bv[32:64], bq[32:64] * s, bz,
         cp["bskip"]])[None]
    wbeta = cp["Wbeta"]
    wa = (wbeta[0:64, 0] + wbeta[128:192, 0])[None]
    wb = (wbeta[64:128, 0] - wbeta[128:192, 0])[None]
    we = cp["We"]
    eye4 = jnp.eye(4, dtype=jnp.float32)
    wp = jnp.stack([jnp.kron(eye4, we[:, 0:32]),
                    jnp.kron(eye4, we[:, 32:64])])
    return wcat, bcat, wa, wb, wp


def kernel(x, edge_attr, edge_index, batch_index, params):
    p = params
    x_pad = jnp.concatenate([x, jnp.zeros((N2 - N,), jnp.int32)])
    off = jnp.searchsorted(
        batch_index, jnp.arange(NG + 1, dtype=jnp.int32)).astype(jnp.int32)
    cnt = (off[1:] - off[:-1]).astype(jnp.float32)[:, None]
    batch_pad = jnp.concatenate(
        [batch_index, jnp.full((NR - N,), NG, jnp.int32)])[:, None]
    src_p = jnp.concatenate(
        [edge_index[0], jnp.zeros((E2 - E,), jnp.int32)])
    dst_p = jnp.concatenate(
        [edge_index[1], jnp.full((E2 - E,), N, jnp.int32)])
    a4 = jnp.concatenate(
        [edge_attr, jnp.zeros((E2 - E, 16), jnp.float32)]).reshape(E2 // 4, 64)
    zacc = jnp.zeros((NR, ACC_W), jnp.float32)

    h = _emb_gather(p["emb"], x_pad)
    scale = jnp.ones((1, FEAT), jnp.float32)
    shift = jnp.zeros((1, FEAT), jnp.float32)

    convs = [p["conv0"]] + list(p["convs"])
    lws = [p["lin0_W"]] + list(p["lins_W"])
    lbs = [p["lin0_b"]] + list(p["lins_b"])
    gs = [p["bn0_g"]] + list(p["bns_g"])
    bs = [p["bn0_b"]] + list(p["bns_b"])

    pools = []
    for l in range(4):
        wcat, bcat, wa, wb, wp = _prep_conv(convs[l])
        tab, xr = _pre(h, scale, shift, wcat, bcat)
        ep = _emat(a4, wp)
        acc3 = _edge(tab.reshape(2 * NR, 128), ep, src_p, dst_p, zacc)
        u, usum, ussq, sc, sh = _post(acc3, xr, wa, wb, lws[l], lbs[l][None],
                                      gs[l][None], bs[l][None])
        scale, shift = sc, sh
        h = u
        if l > 0:
            gmx, gsm = _poolk(u, batch_pad)
            pools.append((gmx, gsm, sc, sh))

    return _mlp(pools, cnt,
                p["lin1_W"], p["lin1_b"], p["lin2_W"], p["lin2_b"],
                p["lin3_W"], p["lin3_b"])
